# baseline (speedup 1.0000x reference)
import jax
import jax.numpy as jnp
from jax.experimental import pallas as pl
from jax.experimental.pallas import tpu as pltpu

T_BLK = 1024
S_BLK = 2048


def _argmin_kernel(tar_ref, src_ref, tsq_ref, ssq_ref, out_ref,
                   bv_ref, bc_ref):
    t = tar_ref[...]
    tsq = tsq_ref[...]
    n_src = src_ref.shape[1]

    def dist(c):
        s = src_ref[:, pl.ds(c * S_BLK, S_BLK)]
        mm2 = jax.lax.dot_general(
            t, s, (((1,), (0,)), ((), ())),
            preferred_element_type=jnp.float32)
        ssq = ssq_ref[:, pl.ds(c * S_BLK, S_BLK)]
        return (tsq + mm2) + ssq

    def pair(c0):
        d2a = dist(c0)
        d2b = dist(c0 + 1)
        lt = d2b < d2a
        local = jnp.minimum(d2a, d2b)
        localc = jnp.where(lt, jnp.float32(1), jnp.float32(0)) + c0.astype(jnp.float32)
        return local, localc

    local, localc = pair(jnp.int32(0))
    bv_ref[...] = local
    bc_ref[...] = localc

    def body(p, _):
        local, localc = pair(2 * p)
        bv = bv_ref[...]
        mask = local < bv
        bv_ref[...] = jnp.where(mask, local, bv)
        bc_ref[...] = jnp.where(mask, localc, bc_ref[...])
        return 0

    jax.lax.fori_loop(1, n_src // (2 * S_BLK), body, 0)

    bv = bv_ref[...]
    vmin = jnp.min(bv, axis=1, keepdims=True)
    lane = jax.lax.broadcasted_iota(jnp.int32, (T_BLK, S_BLK), 1)
    flat = bc_ref[...] * jnp.float32(S_BLK) + lane.astype(jnp.float32)
    cand = jnp.where(bv == vmin, flat, jnp.float32(1e9))
    idx = jnp.min(cand, axis=1)
    out_ref[...] = idx.astype(jnp.int32).reshape(T_BLK, 1)


def _nn_indices_pallas(src_V, tar_V):
    n_src = src_V.shape[0]
    n_tar = tar_V.shape[0]
    tsq = jnp.sum(tar_V * tar_V, axis=1).reshape(n_tar, 1)
    ssq = jnp.sum(src_V * src_V, axis=1).reshape(1, n_src)
    tar_bf = (-2.0 * tar_V).astype(jnp.bfloat16)
    src_bf = src_V.T.astype(jnp.bfloat16)
    idx = pl.pallas_call(
        _argmin_kernel,
        grid=(n_tar // T_BLK,),
        in_specs=[
            pl.BlockSpec((T_BLK, 3), lambda i: (i, 0)),
            pl.BlockSpec((3, n_src), lambda i: (0, 0)),
            pl.BlockSpec((T_BLK, 1), lambda i: (i, 0)),
            pl.BlockSpec((1, n_src), lambda i: (0, 0)),
        ],
        out_specs=pl.BlockSpec((T_BLK, 1), lambda i: (i, 0)),
        out_shape=jax.ShapeDtypeStruct((n_tar, 1), jnp.int32),
        scratch_shapes=[
            pltpu.VMEM((T_BLK, S_BLK), jnp.float32),
            pltpu.VMEM((T_BLK, S_BLK), jnp.float32),
        ],
    )(tar_bf, src_bf, tsq, ssq)
    return idx[:, 0]


def kernel(src_V, tar_V):
    idx = _nn_indices_pallas(src_V, tar_V)
    g = jnp.take(src_V, idx, axis=0) - tar_V
    return 0.5 * jnp.sum(g * g)


# speedup vs baseline: 1.1127x; 1.1127x over previous
import jax
import jax.numpy as jnp
from jax.experimental import pallas as pl
from jax.experimental.pallas import tpu as pltpu

T_BLK = 1024
S_BLK = 4096


def _argmin_kernel(tar_ref, src_ref, tsq_ref, ssq_ref, out_ref,
                   bv_ref, bc_ref):
    t = tar_ref[...]
    tsq = tsq_ref[...]
    n_src = src_ref.shape[1]

    def dist(c):
        s = src_ref[:, pl.ds(c * S_BLK, S_BLK)]
        mm2 = jax.lax.dot_general(
            t, s, (((1,), (0,)), ((), ())),
            preferred_element_type=jnp.float32)
        ssq = ssq_ref[:, pl.ds(c * S_BLK, S_BLK)]
        return (tsq + mm2) + ssq

    bv_ref[...] = dist(jnp.int32(0))
    bc_ref[...] = jnp.zeros((T_BLK, S_BLK), jnp.float32)

    def body(c, _):
        d2 = dist(c)
        bv = bv_ref[...]
        mask = d2 < bv
        bv_ref[...] = jnp.where(mask, d2, bv)
        bc_ref[...] = jnp.where(mask, c.astype(jnp.float32), bc_ref[...])
        return 0

    jax.lax.fori_loop(1, n_src // S_BLK, body, 0)

    bv = bv_ref[...]
    vmin = jnp.min(bv, axis=1, keepdims=True)
    lane = jax.lax.broadcasted_iota(jnp.int32, (T_BLK, S_BLK), 1)
    flat = bc_ref[...] * jnp.float32(S_BLK) + lane.astype(jnp.float32)
    cand = jnp.where(bv == vmin, flat, jnp.float32(1e9))
    idx = jnp.min(cand, axis=1)
    out_ref[...] = idx.astype(jnp.int32).reshape(T_BLK, 1)


def _nn_indices_pallas(src_V, tar_V):
    n_src = src_V.shape[0]
    n_tar = tar_V.shape[0]
    tsq = jnp.sum(tar_V * tar_V, axis=1).reshape(n_tar, 1)
    ssq = jnp.sum(src_V * src_V, axis=1).reshape(1, n_src)
    tar_bf = (-2.0 * tar_V).astype(jnp.bfloat16)
    src_bf = src_V.T.astype(jnp.bfloat16)
    idx = pl.pallas_call(
        _argmin_kernel,
        grid=(n_tar // T_BLK,),
        in_specs=[
            pl.BlockSpec((T_BLK, 3), lambda i: (i, 0)),
            pl.BlockSpec((3, n_src), lambda i: (0, 0)),
            pl.BlockSpec((T_BLK, 1), lambda i: (i, 0)),
            pl.BlockSpec((1, n_src), lambda i: (0, 0)),
        ],
        out_specs=pl.BlockSpec((T_BLK, 1), lambda i: (i, 0)),
        out_shape=jax.ShapeDtypeStruct((n_tar, 1), jnp.int32),
        scratch_shapes=[
            pltpu.VMEM((T_BLK, S_BLK), jnp.float32),
            pltpu.VMEM((T_BLK, S_BLK), jnp.float32),
        ],
    )(tar_bf, src_bf, tsq, ssq)
    return idx[:, 0]


def kernel(src_V, tar_V):
    idx = _nn_indices_pallas(src_V, tar_V)
    g = jnp.take(src_V, idx, axis=0) - tar_V
    return 0.5 * jnp.sum(g * g)
